# Initial kernel scaffold; baseline (speedup 1.0000x reference)
import functools

import jax
import jax.numpy as jnp
from jax import lax
from jax.experimental import pallas as pl
from jax.experimental.pallas import tpu as pltpu
from jax.experimental.pallas import tpu_sc as plsc

M = 32768
K = 256
NE = 8192
BM = 512
BN = 1024
BETA = 0.25

_NC = 2
_NS = 16
_NW = _NC * _NS
_BPW = M // _NW
_CH = 128
_NCH = _BPW // _CH


def _dist_argmin_kernel(z_ref, embt_ref, z2_ref, e2_ref, idx_ref, loss_ref):
    pid = pl.program_id(0)
    z = z_ref[...]
    z2 = z2_ref[...]

    best_d = jnp.full((BM, 1), jnp.inf, dtype=jnp.float32)
    best_i = jnp.zeros((BM, 1), dtype=jnp.int32)
    for c in range(NE // BN):
        e = embt_ref[:, pl.ds(c * BN, BN)]
        s = jnp.dot(z, e, preferred_element_type=jnp.float32)
        e2 = e2_ref[:, pl.ds(c * BN, BN)]
        d = (z2 - 2.0 * s) + e2
        m = jnp.min(d, axis=1, keepdims=True)
        iota = lax.broadcasted_iota(jnp.int32, (BM, BN), 1)
        cand = jnp.min(jnp.where(d == m, iota, NE), axis=1, keepdims=True)
        cand = cand + c * BN
        take = m < best_d
        best_i = jnp.where(take, cand, best_i)
        best_d = jnp.where(take, m, best_d)

    idx_ref[...] = best_i

    @pl.when(pid == 0)
    def _():
        loss_ref[0, 0] = 0.0

    loss_ref[0, 0] += jnp.sum(best_d)

    @pl.when(pid == pl.num_programs(0) - 1)
    def _():
        loss_ref[0, 0] = loss_ref[0, 0] * ((1.0 + BETA) / (M * K))


def _dist_argmin(z, embt, z2, e2):
    grid = (M // BM,)
    return pl.pallas_call(
        _dist_argmin_kernel,
        grid=grid,
        in_specs=[
            pl.BlockSpec((BM, K), lambda m: (m, 0)),
            pl.BlockSpec((K, NE), lambda m: (0, 0)),
            pl.BlockSpec((BM, 1), lambda m: (m, 0)),
            pl.BlockSpec((1, NE), lambda m: (0, 0)),
        ],
        out_specs=[
            pl.BlockSpec((BM, 1), lambda m: (m, 0)),
            pl.BlockSpec((1, 1), lambda m: (0, 0), memory_space=pltpu.SMEM),
        ],
        out_shape=[
            jax.ShapeDtypeStruct((M, 1), jnp.int32),
            jax.ShapeDtypeStruct((1, 1), jnp.float32),
        ],
    )(z, embt, z2, e2)


_sc_mesh = plsc.VectorSubcoreMesh(core_axis_name="c", subcore_axis_name="s")


@functools.partial(
    pl.kernel,
    mesh=_sc_mesh,
    out_type=jax.ShapeDtypeStruct((M, K), jnp.float32),
    scratch_types=[
        pltpu.VMEM((_CH,), jnp.int32),
        pltpu.VMEM((_CH,), jnp.int32),
        pltpu.VMEM((_CH, K), jnp.float32),
        pltpu.VMEM((_CH, K), jnp.float32),
        pltpu.SemaphoreType.DMA,
        pltpu.SemaphoreType.DMA,
    ],
)
def _gather_kernel(emb_hbm, idx_hbm, out_hbm,
                   idx_a, idx_b, rows_a, rows_b, sem_a, sem_b):
    wid = lax.axis_index("s") * _NC + lax.axis_index("c")
    base = wid * _BPW
    idx_bufs = (idx_a, idx_b)
    row_bufs = (rows_a, rows_b)
    sems = (sem_a, sem_b)
    prev = None
    for c in range(_NCH):
        b = c % 2
        pltpu.sync_copy(idx_hbm.at[pl.ds(base + c * _CH, _CH)], idx_bufs[b])
        cur = pltpu.async_copy(emb_hbm.at[idx_bufs[b]], row_bufs[b], sems[b])
        if prev is not None:
            prev.wait()
            pltpu.sync_copy(row_bufs[1 - b],
                            out_hbm.at[pl.ds(base + (c - 1) * _CH, _CH)])
        prev = cur
    prev.wait()
    pltpu.sync_copy(row_bufs[(_NCH - 1) % 2],
                    out_hbm.at[pl.ds(base + (_NCH - 1) * _CH, _CH)])


def kernel(z, emb):
    z2 = jnp.sum(z ** 2, axis=1, keepdims=True)
    e2 = jnp.sum(emb ** 2, axis=1)[None, :]
    embt = emb.T
    idx2d, loss2d = _dist_argmin(z, embt, z2, e2)
    indices = idx2d.reshape(M)
    z_q = _gather_kernel(emb, indices)
    return (z_q, loss2d[0, 0])


# speedup vs baseline: 1.1187x; 1.1187x over previous
import functools

import jax
import jax.numpy as jnp
from jax import lax
from jax.experimental import pallas as pl
from jax.experimental.pallas import tpu as pltpu
from jax.experimental.pallas import tpu_sc as plsc

M = 32768
K = 256
NE = 8192
BM = 512
BN = 1024
BETA = 0.25

_NC = 2
_NS = 16
_NW = _NC * _NS
_BPW = M // _NW
_CH = 128
_NCH = _BPW // _CH


def _dist_argmin_kernel(z_ref, embt_ref, z2_ref, e2_ref, idx_ref, loss_ref):
    pid = pl.program_id(0)
    z = z_ref[...]
    z2 = z2_ref[...]

    best_d = jnp.full((BM, 1), jnp.inf, dtype=jnp.float32)
    best_i = jnp.zeros((BM, 1), dtype=jnp.int32)
    for c in range(NE // BN):
        e = embt_ref[:, pl.ds(c * BN, BN)]
        s = jnp.dot(z, e, preferred_element_type=jnp.float32)
        e2 = e2_ref[:, pl.ds(c * BN, BN)]
        d = (z2 - 2.0 * s) + e2
        m = jnp.min(d, axis=1, keepdims=True)
        iota = lax.broadcasted_iota(jnp.int32, (BM, BN), 1)
        cand = jnp.min(jnp.where(d == m, iota, NE), axis=1, keepdims=True)
        cand = cand + c * BN
        take = m < best_d
        best_i = jnp.where(take, cand, best_i)
        best_d = jnp.where(take, m, best_d)

    idx_ref[...] = best_i

    @pl.when(pid == 0)
    def _():
        loss_ref[0, 0] = 0.0

    loss_ref[0, 0] += jnp.sum(best_d)

    @pl.when(pid == pl.num_programs(0) - 1)
    def _():
        loss_ref[0, 0] = loss_ref[0, 0] * ((1.0 + BETA) / (M * K))


def _dist_argmin(z, embt, z2, e2):
    grid = (M // BM,)
    return pl.pallas_call(
        _dist_argmin_kernel,
        grid=grid,
        in_specs=[
            pl.BlockSpec((BM, K), lambda m: (m, 0)),
            pl.BlockSpec((K, NE), lambda m: (0, 0)),
            pl.BlockSpec((BM, 1), lambda m: (m, 0)),
            pl.BlockSpec((1, NE), lambda m: (0, 0)),
        ],
        out_specs=[
            pl.BlockSpec((BM, 1), lambda m: (m, 0)),
            pl.BlockSpec((1, 1), lambda m: (0, 0), memory_space=pltpu.SMEM),
        ],
        out_shape=[
            jax.ShapeDtypeStruct((M, 1), jnp.int32),
            jax.ShapeDtypeStruct((1, 1), jnp.float32),
        ],
    )(z, embt, z2, e2)


@functools.cache
def _make_gather_kernel():
    mesh = plsc.VectorSubcoreMesh(core_axis_name="c", subcore_axis_name="s")

    @functools.partial(
        pl.kernel,
        mesh=mesh,
        out_type=jax.ShapeDtypeStruct((M, K), jnp.float32),
        scratch_types=[
            pltpu.VMEM((_CH,), jnp.int32),
            pltpu.VMEM((_CH,), jnp.int32),
            pltpu.VMEM((_CH, K), jnp.float32),
            pltpu.VMEM((_CH, K), jnp.float32),
            pltpu.SemaphoreType.DMA,
            pltpu.SemaphoreType.DMA,
        ],
    )
    def gather_kernel(emb_hbm, idx_hbm, out_hbm,
                      idx_a, idx_b, rows_a, rows_b, sem_a, sem_b):
        wid = lax.axis_index("s") * _NC + lax.axis_index("c")
        base = wid * _BPW
        idx_bufs = (idx_a, idx_b)
        row_bufs = (rows_a, rows_b)
        sems = (sem_a, sem_b)
        prev = None
        for c in range(_NCH):
            b = c % 2
            pltpu.sync_copy(idx_hbm.at[pl.ds(base + c * _CH, _CH)], idx_bufs[b])
            cur = pltpu.async_copy(emb_hbm.at[idx_bufs[b]], row_bufs[b], sems[b])
            if prev is not None:
                prev.wait()
                pltpu.sync_copy(row_bufs[1 - b],
                                out_hbm.at[pl.ds(base + (c - 1) * _CH, _CH)])
            prev = cur
        prev.wait()
        pltpu.sync_copy(row_bufs[(_NCH - 1) % 2],
                        out_hbm.at[pl.ds(base + (_NCH - 1) * _CH, _CH)])

    return gather_kernel


def kernel(z, emb):
    z2 = jnp.sum(z ** 2, axis=1, keepdims=True)
    e2 = jnp.sum(emb ** 2, axis=1)[None, :]
    embt = emb.T
    idx2d, loss2d = _dist_argmin(z, embt, z2, e2)
    indices = idx2d.reshape(M)
    z_q = _make_gather_kernel()(emb, indices)
    return (z_q, loss2d[0, 0])


# speedup vs baseline: 1.4470x; 1.2935x over previous
import functools

import jax
import jax.numpy as jnp
from jax import lax
from jax.experimental import pallas as pl
from jax.experimental.pallas import tpu as pltpu
from jax.experimental.pallas import tpu_sc as plsc

M = 32768
K = 256
NE = 8192
BM = 512
BN = 1024
BETA = 0.25

_NC = 2
_NS = 16
_NW = _NC * _NS
_BPW = M // _NW
_CH = 128
_NCH = _BPW // _CH


def _dist_argmin_kernel(z_ref, embt_ref, z2_ref, e2_ref, idx_ref, loss_ref):
    pid = pl.program_id(0)
    z = z_ref[...]
    zd = z + z
    z2 = z2_ref[...]

    best_d = jnp.full((BM, 1), jnp.inf, dtype=jnp.float32)
    best_i = jnp.zeros((BM, 1), dtype=jnp.float32)
    iota_f = lax.broadcasted_iota(jnp.int32, (BM, BN), 1).astype(jnp.float32)
    for c in range(NE // BN):
        e = embt_ref[:, pl.ds(c * BN, BN)]
        s2 = jnp.dot(zd, e, preferred_element_type=jnp.float32)
        e2 = e2_ref[:, pl.ds(c * BN, BN)]
        d = (z2 - s2) + e2
        m = jnp.min(d, axis=1, keepdims=True)
        cand = jnp.min(jnp.where(d == m, iota_f, float(NE)), axis=1, keepdims=True)
        cand = cand + float(c * BN)
        take = m < best_d
        best_i = jnp.where(take, cand, best_i)
        best_d = jnp.where(take, m, best_d)

    idx_ref[...] = best_i.astype(jnp.int32)

    @pl.when(pid == 0)
    def _():
        loss_ref[0, 0] = 0.0

    loss_ref[0, 0] += jnp.sum(best_d)

    @pl.when(pid == pl.num_programs(0) - 1)
    def _():
        loss_ref[0, 0] = loss_ref[0, 0] * ((1.0 + BETA) / (M * K))


def _dist_argmin(z, embt, z2, e2):
    grid = (M // BM,)
    return pl.pallas_call(
        _dist_argmin_kernel,
        grid=grid,
        in_specs=[
            pl.BlockSpec((BM, K), lambda m: (m, 0)),
            pl.BlockSpec((K, NE), lambda m: (0, 0)),
            pl.BlockSpec((BM, 1), lambda m: (m, 0)),
            pl.BlockSpec((1, NE), lambda m: (0, 0)),
        ],
        out_specs=[
            pl.BlockSpec((BM, 1), lambda m: (m, 0)),
            pl.BlockSpec((1, 1), lambda m: (0, 0), memory_space=pltpu.SMEM),
        ],
        out_shape=[
            jax.ShapeDtypeStruct((M, 1), jnp.int32),
            jax.ShapeDtypeStruct((1, 1), jnp.float32),
        ],
    )(z, embt, z2, e2)


@functools.cache
def _make_gather_kernel():
    mesh = plsc.VectorSubcoreMesh(core_axis_name="c", subcore_axis_name="s")

    @functools.partial(
        pl.kernel,
        mesh=mesh,
        out_type=jax.ShapeDtypeStruct((M, K), jnp.float32),
        scratch_types=[
            pltpu.VMEM((_CH,), jnp.int32),
            pltpu.VMEM((_CH,), jnp.int32),
            pltpu.VMEM((_CH, K), jnp.float32),
            pltpu.VMEM((_CH, K), jnp.float32),
            pltpu.SemaphoreType.DMA,
            pltpu.SemaphoreType.DMA,
        ],
    )
    def gather_kernel(emb_hbm, idx_hbm, out_hbm,
                      idx_a, idx_b, rows_a, rows_b, sem_a, sem_b):
        wid = lax.axis_index("s") * _NC + lax.axis_index("c")
        base = wid * _BPW
        idx_bufs = (idx_a, idx_b)
        row_bufs = (rows_a, rows_b)
        sems = (sem_a, sem_b)
        prev = None
        for c in range(_NCH):
            b = c % 2
            pltpu.sync_copy(idx_hbm.at[pl.ds(base + c * _CH, _CH)], idx_bufs[b])
            cur = pltpu.async_copy(emb_hbm.at[idx_bufs[b]], row_bufs[b], sems[b])
            if prev is not None:
                prev.wait()
                pltpu.sync_copy(row_bufs[1 - b],
                                out_hbm.at[pl.ds(base + (c - 1) * _CH, _CH)])
            prev = cur
        prev.wait()
        pltpu.sync_copy(row_bufs[(_NCH - 1) % 2],
                        out_hbm.at[pl.ds(base + (_NCH - 1) * _CH, _CH)])

    return gather_kernel


def kernel(z, emb):
    z2 = jnp.sum(z ** 2, axis=1, keepdims=True)
    e2 = jnp.sum(emb ** 2, axis=1)[None, :]
    embt = emb.T
    idx2d, loss2d = _dist_argmin(z, embt, z2, e2)
    indices = idx2d.reshape(M)
    z_q = _make_gather_kernel()(emb, indices)
    return (z_q, loss2d[0, 0])


# speedup vs baseline: 1.6508x; 1.1408x over previous
import functools

import jax
import jax.numpy as jnp
from jax import lax
from jax.experimental import pallas as pl
from jax.experimental.pallas import tpu as pltpu
from jax.experimental.pallas import tpu_sc as plsc

M = 32768
K = 256
NE = 8192
BM = 1024
BN = 1024
BETA = 0.25

_NC = 2
_NS = 16
_NW = _NC * _NS
_BPW = M // _NW
_CH = 128
_NCH = _BPW // _CH


def _dist_argmin_kernel(z_ref, emb_ref, e2_ref, idx_ref, loss_ref):
    pid = pl.program_id(0)
    z = z_ref[...]
    zd = z + z
    z2 = jnp.sum(z * z, axis=1, keepdims=True)

    best_d = jnp.full((BM, 1), jnp.inf, dtype=jnp.float32)
    best_i = jnp.zeros((BM, 1), dtype=jnp.float32)
    iota_f = lax.broadcasted_iota(jnp.int32, (BM, BN), 1).astype(jnp.float32)
    for c in range(NE // BN):
        e = emb_ref[pl.ds(c * BN, BN), :]
        s2 = lax.dot_general(zd, e, (((1,), (1,)), ((), ())),
                             preferred_element_type=jnp.float32)
        e2 = e2_ref[:, pl.ds(c * BN, BN)]
        d = (z2 - s2) + e2
        m = jnp.min(d, axis=1, keepdims=True)
        cand = jnp.min(jnp.where(d == m, iota_f, float(NE)), axis=1, keepdims=True)
        cand = cand + float(c * BN)
        take = m < best_d
        best_i = jnp.where(take, cand, best_i)
        best_d = jnp.where(take, m, best_d)

    idx_ref[...] = best_i.astype(jnp.int32)

    @pl.when(pid == 0)
    def _():
        loss_ref[0, 0] = 0.0

    loss_ref[0, 0] += jnp.sum(best_d)

    @pl.when(pid == pl.num_programs(0) - 1)
    def _():
        loss_ref[0, 0] = loss_ref[0, 0] * ((1.0 + BETA) / (M * K))


def _dist_argmin(z, emb, e2):
    grid = (M // BM,)
    return pl.pallas_call(
        _dist_argmin_kernel,
        grid=grid,
        in_specs=[
            pl.BlockSpec((BM, K), lambda m: (m, 0)),
            pl.BlockSpec((NE, K), lambda m: (0, 0)),
            pl.BlockSpec((1, NE), lambda m: (0, 0)),
        ],
        out_specs=[
            pl.BlockSpec((BM, 1), lambda m: (m, 0)),
            pl.BlockSpec((1, 1), lambda m: (0, 0), memory_space=pltpu.SMEM),
        ],
        out_shape=[
            jax.ShapeDtypeStruct((M, 1), jnp.int32),
            jax.ShapeDtypeStruct((1, 1), jnp.float32),
        ],
    )(z, emb, e2)


@functools.cache
def _make_gather_kernel():
    mesh = plsc.VectorSubcoreMesh(core_axis_name="c", subcore_axis_name="s")

    @functools.partial(
        pl.kernel,
        mesh=mesh,
        out_type=jax.ShapeDtypeStruct((M, K), jnp.float32),
        scratch_types=[
            pltpu.VMEM((_CH,), jnp.int32),
            pltpu.VMEM((_CH,), jnp.int32),
            pltpu.VMEM((_CH, K), jnp.float32),
            pltpu.VMEM((_CH, K), jnp.float32),
            pltpu.SemaphoreType.DMA,
            pltpu.SemaphoreType.DMA,
        ],
    )
    def gather_kernel(emb_hbm, idx_hbm, out_hbm,
                      idx_a, idx_b, rows_a, rows_b, sem_a, sem_b):
        wid = lax.axis_index("s") * _NC + lax.axis_index("c")
        base = wid * _BPW
        idx_bufs = (idx_a, idx_b)
        row_bufs = (rows_a, rows_b)
        sems = (sem_a, sem_b)
        prev = None
        for c in range(_NCH):
            b = c % 2
            pltpu.sync_copy(idx_hbm.at[pl.ds(base + c * _CH, _CH)], idx_bufs[b])
            cur = pltpu.async_copy(emb_hbm.at[idx_bufs[b]], row_bufs[b], sems[b])
            if prev is not None:
                prev.wait()
                pltpu.sync_copy(row_bufs[1 - b],
                                out_hbm.at[pl.ds(base + (c - 1) * _CH, _CH)])
            prev = cur
        prev.wait()
        pltpu.sync_copy(row_bufs[(_NCH - 1) % 2],
                        out_hbm.at[pl.ds(base + (_NCH - 1) * _CH, _CH)])

    return gather_kernel


def kernel(z, emb):
    e2 = jnp.sum(emb ** 2, axis=1)[None, :]
    idx2d, loss2d = _dist_argmin(z, emb, e2)
    indices = idx2d.reshape(M)
    z_q = _make_gather_kernel()(emb, indices)
    return (z_q, loss2d[0, 0])


# speedup vs baseline: 1.6703x; 1.0118x over previous
import functools

import jax
import jax.numpy as jnp
from jax import lax
from jax.experimental import pallas as pl
from jax.experimental.pallas import tpu as pltpu
from jax.experimental.pallas import tpu_sc as plsc

M = 32768
K = 256
NE = 8192
BM = 2048
BN = 1024
BETA = 0.25

_NC = 2
_NS = 16
_NW = _NC * _NS
_BPW = M // _NW
_CH = 128
_NCH = _BPW // _CH


def _dist_argmin_kernel(z_ref, emb_ref, e2_ref, idx_ref, loss_ref):
    pid = pl.program_id(0)
    z = z_ref[...]
    zd = z + z
    z2 = jnp.sum(z * z, axis=1, keepdims=True)

    best_d = jnp.full((BM, 1), jnp.inf, dtype=jnp.float32)
    best_i = jnp.zeros((BM, 1), dtype=jnp.float32)
    iota_f = lax.broadcasted_iota(jnp.int32, (BM, BN), 1).astype(jnp.float32)
    for c in range(NE // BN):
        e = emb_ref[pl.ds(c * BN, BN), :]
        s2 = lax.dot_general(zd, e, (((1,), (1,)), ((), ())),
                             preferred_element_type=jnp.float32)
        e2 = e2_ref[:, pl.ds(c * BN, BN)]
        d = (z2 - s2) + e2
        m = jnp.min(d, axis=1, keepdims=True)
        cand = jnp.min(jnp.where(d == m, iota_f, float(NE)), axis=1, keepdims=True)
        cand = cand + float(c * BN)
        take = m < best_d
        best_i = jnp.where(take, cand, best_i)
        best_d = jnp.where(take, m, best_d)

    idx_ref[...] = best_i.astype(jnp.int32)

    @pl.when(pid == 0)
    def _():
        loss_ref[0, 0] = 0.0

    loss_ref[0, 0] += jnp.sum(best_d)

    @pl.when(pid == pl.num_programs(0) - 1)
    def _():
        loss_ref[0, 0] = loss_ref[0, 0] * ((1.0 + BETA) / (M * K))


def _dist_argmin(z, emb, e2):
    grid = (M // BM,)
    return pl.pallas_call(
        _dist_argmin_kernel,
        grid=grid,
        in_specs=[
            pl.BlockSpec((BM, K), lambda m: (m, 0)),
            pl.BlockSpec((NE, K), lambda m: (0, 0)),
            pl.BlockSpec((1, NE), lambda m: (0, 0)),
        ],
        out_specs=[
            pl.BlockSpec((BM, 1), lambda m: (m, 0)),
            pl.BlockSpec((1, 1), lambda m: (0, 0), memory_space=pltpu.SMEM),
        ],
        out_shape=[
            jax.ShapeDtypeStruct((M, 1), jnp.int32),
            jax.ShapeDtypeStruct((1, 1), jnp.float32),
        ],
    )(z, emb, e2)


@functools.cache
def _make_gather_kernel():
    mesh = plsc.VectorSubcoreMesh(core_axis_name="c", subcore_axis_name="s")

    @functools.partial(
        pl.kernel,
        mesh=mesh,
        out_type=jax.ShapeDtypeStruct((M, K), jnp.float32),
        scratch_types=[
            pltpu.VMEM((_CH,), jnp.int32),
            pltpu.VMEM((_CH,), jnp.int32),
            pltpu.VMEM((_CH, K), jnp.float32),
            pltpu.VMEM((_CH, K), jnp.float32),
            pltpu.SemaphoreType.DMA,
            pltpu.SemaphoreType.DMA,
        ],
    )
    def gather_kernel(emb_hbm, idx_hbm, out_hbm,
                      idx_a, idx_b, rows_a, rows_b, sem_a, sem_b):
        wid = lax.axis_index("s") * _NC + lax.axis_index("c")
        base = wid * _BPW
        idx_bufs = (idx_a, idx_b)
        row_bufs = (rows_a, rows_b)
        sems = (sem_a, sem_b)
        prev = None
        for c in range(_NCH):
            b = c % 2
            pltpu.sync_copy(idx_hbm.at[pl.ds(base + c * _CH, _CH)], idx_bufs[b])
            cur = pltpu.async_copy(emb_hbm.at[idx_bufs[b]], row_bufs[b], sems[b])
            if prev is not None:
                prev.wait()
                pltpu.sync_copy(row_bufs[1 - b],
                                out_hbm.at[pl.ds(base + (c - 1) * _CH, _CH)])
            prev = cur
        prev.wait()
        pltpu.sync_copy(row_bufs[(_NCH - 1) % 2],
                        out_hbm.at[pl.ds(base + (_NCH - 1) * _CH, _CH)])

    return gather_kernel


def kernel(z, emb):
    e2 = jnp.sum(emb ** 2, axis=1)[None, :]
    idx2d, loss2d = _dist_argmin(z, emb, e2)
    indices = idx2d.reshape(M)
    z_q = _make_gather_kernel()(emb, indices)
    return (z_q, loss2d[0, 0])
